# baseline (speedup 1.0000x reference)
import functools

import jax
import jax.numpy as jnp
from jax import lax
from jax.experimental import pallas as pl
from jax.experimental.pallas import tpu as pltpu
from jax.experimental.pallas import tpu_sc as plsc

_LANES = 16


def _sc_info():
    try:
        info = plsc.get_sparse_core_info()
        return info.num_cores, info.num_subcores
    except Exception:
        return 2, 16


@functools.lru_cache(maxsize=None)
def _build(B, S, D, V):
    NC, NS = _sc_info()
    NW = NC * NS
    assert S % NW == 0
    SBLK = S // NW
    CHUNK = 32
    assert SBLK % CHUNK == 0
    halves = SBLK // CHUNK
    NBUF = 3
    n_vec = D // _LANES
    n_chunks = B * halves

    mesh = plsc.VectorSubcoreMesh(core_axis_name="c", subcore_axis_name="s")

    @functools.partial(
        pl.kernel,
        out_type=jax.ShapeDtypeStruct((B, S, D), jnp.float32),
        mesh=mesh,
        scratch_types=[
            pltpu.VMEM((B, SBLK), jnp.int32),
            pltpu.VMEM((SBLK, D), jnp.float32),
            [pltpu.VMEM((CHUNK, D), jnp.float32) for _ in range(NBUF)],
            pltpu.SemaphoreType.DMA,
            [pltpu.SemaphoreType.DMA for _ in range(B)],
            [pltpu.SemaphoreType.DMA for _ in range(NBUF)],
            [pltpu.SemaphoreType.DMA for _ in range(NBUF)],
        ],
    )
    def k(x_hbm, table_hbm, pos_hbm, out_hbm, idx_v, pos_v, rows, sem_p, sem_i, sem_g, sem_o):
        wid = lax.axis_index("s") * NC + lax.axis_index("c")
        s_base = wid * SBLK

        p_desc = pltpu.async_copy(pos_hbm.at[pl.ds(s_base, SBLK)], pos_v, sem_p)
        i_descs = [
            pltpu.async_copy(x_hbm.at[b, pl.ds(s_base, SBLK)], idx_v.at[b], sem_i[b])
            for b in range(B)
        ]
        idx_ready = [False] * B

        def gather(c):
            b, h = divmod(c, halves)
            if not idx_ready[b]:
                i_descs[b].wait()
                idx_ready[b] = True
            return pltpu.async_copy(
                table_hbm.at[idx_v.at[b, pl.ds(h * CHUNK, CHUNK)]],
                rows[c % NBUF],
                sem_g[c % NBUF],
            )

        g_descs = [None] * n_chunks
        o_descs = [None] * n_chunks
        for c in range(min(NBUF - 1, n_chunks)):
            g_descs[c] = gather(c)
        p_desc.wait()
        for c in range(n_chunks):
            buf = rows[c % NBUF]
            b, h = divmod(c, halves)
            g_descs[c].wait()

            @plsc.parallel_loop(0, CHUNK, 1, unroll=4)
            def _add(r):
                pr = h * CHUNK + r
                for v in range(n_vec):
                    sl = pl.ds(v * _LANES, _LANES)
                    plsc.addupdate(buf.at[r, sl], pos_v[pr, sl])

            o_descs[c] = pltpu.async_copy(
                buf,
                out_hbm.at[b, pl.ds(s_base + h * CHUNK, CHUNK)],
                sem_o[c % NBUF],
            )
            nxt = c + NBUF - 1
            if nxt < n_chunks and g_descs[nxt] is None:
                if c > 0:
                    o_descs[c - 1].wait()
                g_descs[nxt] = gather(nxt)
        for c in range(n_chunks):
            if o_descs[c] is not None and c >= n_chunks - NBUF:
                o_descs[c].wait()

    return k


def kernel(x, table, pos_enc):
    B, S = x.shape
    V, D = table.shape
    if x.dtype != jnp.int32:
        x = x.astype(jnp.int32)
    k = _build(B, S, D, V)
    return k(x, table, pos_enc)


# speedup vs baseline: 1.1487x; 1.1487x over previous
import functools

import jax
import jax.numpy as jnp
from jax import lax
from jax.experimental import pallas as pl
from jax.experimental.pallas import tpu as pltpu
from jax.experimental.pallas import tpu_sc as plsc

_LANES = 16


def _sc_info():
    try:
        info = plsc.get_sparse_core_info()
        return info.num_cores, info.num_subcores
    except Exception:
        return 2, 16


@functools.lru_cache(maxsize=None)
def _build(B, S, D, V):
    NC, NS = _sc_info()
    NW = NC * NS
    assert S % NW == 0
    SBLK = S // NW
    CHUNK = 32
    assert SBLK % CHUNK == 0
    halves = SBLK // CHUNK
    NBUF = 3
    n_vec = D // _LANES
    n_chunks = B * halves

    mesh = plsc.VectorSubcoreMesh(core_axis_name="c", subcore_axis_name="s")

    @functools.partial(
        pl.kernel,
        out_type=jax.ShapeDtypeStruct((B, S, D), jnp.float32),
        mesh=mesh,
        scratch_types=[
            pltpu.VMEM((B, SBLK), jnp.int32),
            pltpu.VMEM((SBLK, D), jnp.float32),
            [pltpu.VMEM((CHUNK, D), jnp.float32) for _ in range(NBUF)],
            pltpu.SemaphoreType.DMA,
            [pltpu.SemaphoreType.DMA for _ in range(B)],
            [pltpu.SemaphoreType.DMA for _ in range(NBUF)],
            [pltpu.SemaphoreType.DMA for _ in range(NBUF)],
        ],
    )
    def k(x_hbm, table_hbm, pos_hbm, out_hbm, idx_v, pos_v, rows, sem_p, sem_i, sem_g, sem_o):
        wid = lax.axis_index("s") * NC + lax.axis_index("c")
        s_base = wid * SBLK

        p_desc = pltpu.async_copy(pos_hbm.at[pl.ds(s_base, SBLK)], pos_v, sem_p)
        i_descs = [
            pltpu.async_copy(x_hbm.at[b, pl.ds(s_base, SBLK)], idx_v.at[b], sem_i[b])
            for b in range(B)
        ]
        idx_ready = [False] * B

        def gather(c):
            b, h = divmod(c, halves)
            if not idx_ready[b]:
                i_descs[b].wait()
                idx_ready[b] = True
            return pltpu.async_copy(
                table_hbm.at[idx_v.at[b, pl.ds(h * CHUNK, CHUNK)]],
                rows[c % NBUF],
                sem_g[c % NBUF],
            )

        g_descs = [None] * n_chunks
        o_descs = [None] * n_chunks
        for c in range(min(NBUF - 1, n_chunks)):
            g_descs[c] = gather(c)
        p_desc.wait()
        for c in range(n_chunks):
            buf = rows[c % NBUF]
            b, h = divmod(c, halves)
            g_descs[c].wait()

            @plsc.parallel_loop(0, CHUNK, 1, unroll=1)
            def _add(r):
                pr = h * CHUNK + r
                for v in range(n_vec):
                    sl = pl.ds(v * _LANES, _LANES)
                    plsc.addupdate(buf.at[r, sl], pos_v[pr, sl])

            o_descs[c] = pltpu.async_copy(
                buf,
                out_hbm.at[b, pl.ds(s_base + h * CHUNK, CHUNK)],
                sem_o[c % NBUF],
            )
            nxt = c + NBUF - 1
            if nxt < n_chunks and g_descs[nxt] is None:
                if c > 0:
                    o_descs[c - 1].wait()
                g_descs[nxt] = gather(nxt)
        for c in range(n_chunks):
            if o_descs[c] is not None and c >= n_chunks - NBUF:
                o_descs[c].wait()

    return k


def kernel(x, table, pos_enc):
    B, S = x.shape
    V, D = table.shape
    if x.dtype != jnp.int32:
        x = x.astype(jnp.int32)
    k = _build(B, S, D, V)
    return k(x, table, pos_enc)
